# baseline (device time: 41105 ns/iter reference)
import jax
import jax.numpy as jnp
from jax import lax
from jax.experimental import pallas as pl
from jax.experimental.pallas import tpu as pltpu

N_DEV = 8
MASKS = (1, 3, 4)
ORDERS = ((0, 1, 2), (1, 2, 0), (2, 0, 1))
COLS = ((0, 384), (384, 384), (768, 256))
RS0_OFF, RS1_OFF, MID_OFF = 0, 512, 768


def _exchange(acc, dst_ref, send_sem, recv_sem, row_off, rows, cols, partner):
    c0, cn = cols
    rdma = pltpu.make_async_remote_copy(
        src_ref=acc.at[pl.ds(row_off, rows), pl.ds(c0, cn)],
        dst_ref=dst_ref,
        send_sem=send_sem,
        recv_sem=recv_sem,
        device_id=(partner,),
        device_id_type=pl.DeviceIdType.MESH,
    )
    rdma.start()
    return rdma


def kernel(ids, E):
    V_loc, D = E.shape
    T = ids.shape[0]

    my = lax.axis_index("i")
    local = ids - my * V_loc
    owned = jnp.logical_and(local >= 0, local < V_loc)
    cs = jnp.cumsum(owned.astype(jnp.int32))
    count = cs[T - 1:]
    slot = jnp.where(owned, cs - 1, T)
    packed_vals = jnp.arange(T, dtype=jnp.int32) | (
        jnp.clip(local, 0, V_loc - 1) << 10)
    packed = jnp.zeros((T,), jnp.int32).at[slot].set(packed_vals, mode="drop")

    def body(pk_ref, n_ref, e_ref, out_ref, stage, acc,
             rs0, rs1, rs2, g_sem, send_sems, rs_recv, ag_recv):
        my_i = lax.axis_index("i")
        b0 = lax.rem(my_i, 2)
        b1 = lax.rem(lax.div(my_i, 2), 2)
        b2 = lax.div(my_i, 4)
        coords = (b0 ^ b1, b1, b2)
        rs_bufs = (rs0, rs1, rs2)
        n = n_ref[0]

        stage[...] = jnp.zeros((T, D), jnp.float32)

        def issue(t, c):
            v = pk_ref[t]
            pltpu.make_async_copy(
                e_ref.at[pl.ds(v >> 10, 1), :],
                stage.at[pl.ds(v & 1023, 1), :],
                g_sem,
            ).start()
            return c

        lax.fori_loop(0, n, issue, jnp.int32(0))

        barrier = pltpu.get_barrier_semaphore()
        for m in MASKS:
            pl.semaphore_signal(
                barrier, inc=1,
                device_id=(my_i ^ m,), device_id_type=pl.DeviceIdType.MESH,
            )
        pl.semaphore_wait(barrier, 3)

        def drain(t, c):
            pltpu.make_async_copy(
                e_ref.at[pl.ds(0, 1), :],
                stage.at[pl.ds(0, 1), :],
                g_sem,
            ).wait()
            return c

        lax.fori_loop(0, n, drain, jnp.int32(0))

        acc[...] = stage[...].astype(jnp.bfloat16)

        def add(p, dst_off, buf_off, rows):
            c0, cn = COLS[p]
            acc[pl.ds(dst_off, rows), pl.ds(c0, cn)] = (
                acc[pl.ds(dst_off, rows), pl.ds(c0, cn)]
                + rs_bufs[p][pl.ds(buf_off, rows), :]
            )

        zero = my_i * 0
        bit = [coords[ORDERS[p][0]] for p in range(3)]

        ex = [
            _exchange(acc, rs_bufs[p].at[pl.ds(RS0_OFF, 512), :],
                      send_sems.at[p], rs_recv.at[0, p],
                      zero + (1 - bit[p]) * 512, 512, COLS[p],
                      my_i ^ MASKS[ORDERS[p][0]])
            for p in range(3)
        ]
        k0 = [zero + bit[p] * 512 for p in range(3)]
        bit1 = [coords[ORDERS[p][1]] for p in range(3)]
        k1 = [k0[p] + bit1[p] * 256 for p in range(3)]

        ex1 = [None] * 3
        for p in range(3):
            ex[p].wait()
            send_off = k0[p] + (1 - bit1[p]) * 256
            add(p, send_off, RS0_OFF + (send_off - k0[p]), 256)
            ex1[p] = _exchange(acc, rs_bufs[p].at[pl.ds(RS1_OFF, 256), :],
                               send_sems.at[p], rs_recv.at[1, p],
                               send_off, 256, COLS[p],
                               my_i ^ MASKS[ORDERS[p][1]])
            add(p, k1[p], RS0_OFF + (k1[p] - k0[p]), 256)

        exm = [None] * 3
        for p in range(3):
            ex1[p].wait()
            add(p, k1[p], RS1_OFF, 256)
            exm[p] = _exchange(acc, rs_bufs[p].at[pl.ds(MID_OFF, 256), :],
                               send_sems.at[p], rs_recv.at[2, p],
                               k1[p], 256, COLS[p],
                               my_i ^ MASKS[ORDERS[p][2]])

        exa1 = [None] * 3
        for p in range(3):
            exm[p].wait()
            add(p, k1[p], MID_OFF, 256)
            c0, cn = COLS[p]
            exa1[p] = _exchange(
                acc, acc.at[pl.ds(k1[p], 256), pl.ds(c0, cn)],
                send_sems.at[p], ag_recv.at[1, p],
                k1[p], 256, COLS[p], my_i ^ MASKS[ORDERS[p][1]])

        exa0 = [None] * 3
        for p in range(3):
            exa1[p].wait()
            c0, cn = COLS[p]
            exa0[p] = _exchange(
                acc, acc.at[pl.ds(k0[p], 512), pl.ds(c0, cn)],
                send_sems.at[p], ag_recv.at[0, p],
                k0[p], 512, COLS[p], my_i ^ MASKS[ORDERS[p][0]])
        for p in range(3):
            c0, cn = COLS[p]
            out_ref[pl.ds(k0[p], 512), pl.ds(c0, cn)] = (
                acc[pl.ds(k0[p], 512), pl.ds(c0, cn)].astype(jnp.float32)
            )
        for p in range(3):
            exa0[p].wait()
            c0, cn = COLS[p]
            o = zero + (1 - bit[p]) * 512
            out_ref[pl.ds(o, 512), pl.ds(c0, cn)] = (
                acc[pl.ds(o, 512), pl.ds(c0, cn)].astype(jnp.float32)
            )

    return pl.pallas_call(
        body,
        out_shape=jax.ShapeDtypeStruct((T, D), jnp.float32),
        in_specs=[
            pl.BlockSpec(memory_space=pltpu.MemorySpace.SMEM),
            pl.BlockSpec(memory_space=pltpu.MemorySpace.SMEM),
            pl.BlockSpec(memory_space=pltpu.MemorySpace.HBM),
        ],
        out_specs=pl.BlockSpec(memory_space=pltpu.MemorySpace.VMEM),
        scratch_shapes=[
            pltpu.VMEM((1024, 1024), jnp.float32),
            pltpu.VMEM((1024, 1024), jnp.bfloat16),
            pltpu.VMEM((1024, 384), jnp.bfloat16),
            pltpu.VMEM((1024, 384), jnp.bfloat16),
            pltpu.VMEM((1024, 256), jnp.bfloat16),
            pltpu.SemaphoreType.DMA,
            pltpu.SemaphoreType.DMA((3,)),
            pltpu.SemaphoreType.DMA((3, 3)),
            pltpu.SemaphoreType.DMA((2, 3)),
        ],
        compiler_params=pltpu.CompilerParams(collective_id=0),
    )(packed, count, E)


# device time: 40414 ns/iter; 1.0171x vs baseline; 1.0171x over previous
import jax
import jax.numpy as jnp
from jax import lax
from jax.experimental import pallas as pl
from jax.experimental.pallas import tpu as pltpu

N_DEV = 8
MASKS = (1, 3, 4)
ORDERS = ((0, 1, 2), (1, 2, 0), (2, 0, 1))
COLS = ((0, 384), (384, 384), (768, 256))
RS0_OFF, RS1_OFF, MID_OFF = 0, 512, 768


def _exchange(acc, dst_ref, send_sem, recv_sem, row_off, rows, cols, partner):
    c0, cn = cols
    rdma = pltpu.make_async_remote_copy(
        src_ref=acc.at[pl.ds(row_off, rows), pl.ds(c0, cn)],
        dst_ref=dst_ref,
        send_sem=send_sem,
        recv_sem=recv_sem,
        device_id=(partner,),
        device_id_type=pl.DeviceIdType.MESH,
    )
    rdma.start()
    return rdma


def kernel(ids, E):
    V_loc, D = E.shape
    T = ids.shape[0]

    my = lax.axis_index("i")
    local = ids - my * V_loc
    owned = jnp.logical_and(local >= 0, local < V_loc)
    owned_i = owned.astype(jnp.int32)
    count = jnp.sum(owned_i).reshape(1)
    slot = jnp.where(owned, jnp.cumsum(owned_i) - 1, T)
    packed_vals = jnp.arange(T, dtype=jnp.int32) | (
        jnp.clip(local, 0, V_loc - 1) << 10)
    packed = jnp.zeros((T,), jnp.int32).at[slot].set(packed_vals, mode="drop")

    def body(pk_ref, n_ref, e_ref, out_ref, stage, acc,
             rs0, rs1, rs2, g_sem, send_sems, rs_recv, ag_recv):
        my_i = lax.axis_index("i")
        b0 = lax.rem(my_i, 2)
        b1 = lax.rem(lax.div(my_i, 2), 2)
        b2 = lax.div(my_i, 4)
        coords = (b0 ^ b1, b1, b2)
        rs_bufs = (rs0, rs1, rs2)
        n = n_ref[0]

        stage[...] = jnp.zeros((T, D), jnp.float32)

        def issue(t, c):
            v = pk_ref[t]
            pltpu.make_async_copy(
                e_ref.at[pl.ds(v >> 10, 1), :],
                stage.at[pl.ds(v & 1023, 1), :],
                g_sem,
            ).start()
            return c

        lax.fori_loop(0, n, issue, jnp.int32(0))

        barrier = pltpu.get_barrier_semaphore()
        for m in MASKS:
            pl.semaphore_signal(
                barrier, inc=1,
                device_id=(my_i ^ m,), device_id_type=pl.DeviceIdType.MESH,
            )
        pl.semaphore_wait(barrier, 3)

        def drain(t, c):
            pltpu.make_async_copy(
                e_ref.at[pl.ds(0, 1), :],
                stage.at[pl.ds(0, 1), :],
                g_sem,
            ).wait()
            return c

        lax.fori_loop(0, n, drain, jnp.int32(0))

        acc[...] = stage[...].astype(jnp.bfloat16)

        def add(p, dst_off, buf_off, rows):
            c0, cn = COLS[p]
            acc[pl.ds(dst_off, rows), pl.ds(c0, cn)] = (
                acc[pl.ds(dst_off, rows), pl.ds(c0, cn)]
                + rs_bufs[p][pl.ds(buf_off, rows), :]
            )

        zero = my_i * 0
        bit = [coords[ORDERS[p][0]] for p in range(3)]

        ex = [
            _exchange(acc, rs_bufs[p].at[pl.ds(RS0_OFF, 512), :],
                      send_sems.at[p], rs_recv.at[0, p],
                      zero + (1 - bit[p]) * 512, 512, COLS[p],
                      my_i ^ MASKS[ORDERS[p][0]])
            for p in range(3)
        ]
        k0 = [zero + bit[p] * 512 for p in range(3)]
        bit1 = [coords[ORDERS[p][1]] for p in range(3)]
        k1 = [k0[p] + bit1[p] * 256 for p in range(3)]

        ex1 = [None] * 3
        for p in range(3):
            ex[p].wait()
            send_off = k0[p] + (1 - bit1[p]) * 256
            add(p, send_off, RS0_OFF + (send_off - k0[p]), 256)
            ex1[p] = _exchange(acc, rs_bufs[p].at[pl.ds(RS1_OFF, 256), :],
                               send_sems.at[p], rs_recv.at[1, p],
                               send_off, 256, COLS[p],
                               my_i ^ MASKS[ORDERS[p][1]])
            add(p, k1[p], RS0_OFF + (k1[p] - k0[p]), 256)

        exm = [None] * 3
        for p in range(3):
            ex1[p].wait()
            add(p, k1[p], RS1_OFF, 256)
            exm[p] = _exchange(acc, rs_bufs[p].at[pl.ds(MID_OFF, 256), :],
                               send_sems.at[p], rs_recv.at[2, p],
                               k1[p], 256, COLS[p],
                               my_i ^ MASKS[ORDERS[p][2]])

        exa1 = [None] * 3
        for p in range(3):
            exm[p].wait()
            add(p, k1[p], MID_OFF, 256)
            c0, cn = COLS[p]
            exa1[p] = _exchange(
                acc, acc.at[pl.ds(k1[p], 256), pl.ds(c0, cn)],
                send_sems.at[p], ag_recv.at[1, p],
                k1[p], 256, COLS[p], my_i ^ MASKS[ORDERS[p][1]])

        exa0 = [None] * 3
        for p in range(3):
            exa1[p].wait()
            c0, cn = COLS[p]
            exa0[p] = _exchange(
                acc, acc.at[pl.ds(k0[p], 512), pl.ds(c0, cn)],
                send_sems.at[p], ag_recv.at[0, p],
                k0[p], 512, COLS[p], my_i ^ MASKS[ORDERS[p][0]])
        for p in range(3):
            c0, cn = COLS[p]
            out_ref[pl.ds(k0[p], 512), pl.ds(c0, cn)] = (
                acc[pl.ds(k0[p], 512), pl.ds(c0, cn)].astype(jnp.float32)
            )
        for p in range(3):
            exa0[p].wait()
            c0, cn = COLS[p]
            o = zero + (1 - bit[p]) * 512
            out_ref[pl.ds(o, 512), pl.ds(c0, cn)] = (
                acc[pl.ds(o, 512), pl.ds(c0, cn)].astype(jnp.float32)
            )

    return pl.pallas_call(
        body,
        out_shape=jax.ShapeDtypeStruct((T, D), jnp.float32),
        in_specs=[
            pl.BlockSpec(memory_space=pltpu.MemorySpace.SMEM),
            pl.BlockSpec(memory_space=pltpu.MemorySpace.SMEM),
            pl.BlockSpec(memory_space=pltpu.MemorySpace.HBM),
        ],
        out_specs=pl.BlockSpec(memory_space=pltpu.MemorySpace.VMEM),
        scratch_shapes=[
            pltpu.VMEM((1024, 1024), jnp.float32),
            pltpu.VMEM((1024, 1024), jnp.bfloat16),
            pltpu.VMEM((1024, 384), jnp.bfloat16),
            pltpu.VMEM((1024, 384), jnp.bfloat16),
            pltpu.VMEM((1024, 256), jnp.bfloat16),
            pltpu.SemaphoreType.DMA,
            pltpu.SemaphoreType.DMA((3,)),
            pltpu.SemaphoreType.DMA((3, 3)),
            pltpu.SemaphoreType.DMA((2, 3)),
        ],
        compiler_params=pltpu.CompilerParams(collective_id=0),
    )(packed, count, E)
